# baseline (speedup 1.0000x reference)
import functools

import jax
import jax.numpy as jnp
from jax import lax
from jax.experimental import pallas as pl
from jax.experimental.pallas import tpu as pltpu
from jax.experimental.pallas import tpu_sc as plsc

B = 16384
DU = 64
DI = 32
FU = 128
FI = 128
FC = 128
NUM_TASKS = 4
CROSS_OUT = 128
LANES = 128

NC = 2
NS = 16
NW = NC * NS
BPW = B // NW
HALF = BPW // 2
IDX_CHUNK = 128

BB = 2048


def _sc_gather_body(user_tab, uid, item_tab, iid, ue_out, ie_out,
                    uidx_v, iidx_v, urows_v, irows_v, usem, isem):
    wid = lax.axis_index("s") * NC + lax.axis_index("c")
    base = wid * BPW
    pltpu.sync_copy(uid.at[pl.ds(base, BPW)], uidx_v)
    pltpu.sync_copy(iid.at[pl.ds(base, BPW)], iidx_v)
    for i in range(BPW // 16):
        s = pl.ds(i * 16, 16)
        uidx_v[s] = lax.shift_right_logical(uidx_v[s], 1)
        iidx_v[s] = lax.shift_right_logical(iidx_v[s], 2)
    for h in range(BPW // HALF):
        copies = []
        for j in range(HALF // IDX_CHUNK):
            src = pl.ds(h * HALF + j * IDX_CHUNK, IDX_CHUNK)
            dst = pl.ds(j * IDX_CHUNK, IDX_CHUNK)
            copies.append(pltpu.async_copy(
                user_tab.at[uidx_v.at[src]], urows_v.at[dst], usem))
            copies.append(pltpu.async_copy(
                item_tab.at[iidx_v.at[src]], irows_v.at[dst], isem))
        for c in copies:
            c.wait()
        out = pl.ds(base + h * HALF, HALF)
        pltpu.sync_copy(urows_v, ue_out.at[out])
        pltpu.sync_copy(irows_v, ie_out.at[out])


_sc_gather = pl.kernel(
    _sc_gather_body,
    out_type=(
        jax.ShapeDtypeStruct((B, LANES), jnp.float32),
        jax.ShapeDtypeStruct((B, LANES), jnp.float32),
    ),
    mesh=plsc.VectorSubcoreMesh(core_axis_name="c", subcore_axis_name="s"),
    scratch_types=[
        pltpu.VMEM((BPW,), jnp.int32),
        pltpu.VMEM((BPW,), jnp.int32),
        pltpu.VMEM((HALF, LANES), jnp.float32),
        pltpu.VMEM((HALF, LANES), jnp.float32),
        pltpu.SemaphoreType.DMA,
        pltpu.SemaphoreType.DMA,
    ],
)


def _dense_body(uf, itf, cf, uw, iw, uid, iid, wu, wi, wc, wt, bu, bi, bc,
                bt, out):
    f32 = jnp.float32
    dot = functools.partial(jnp.dot, preferred_element_type=f32)
    uft = dot(uf[...], wu[...]) + bu[...]
    ift = dot(itf[...], wi[...]) + bi[...]
    cft = dot(cf[...], wc[...]) + bc[...]
    wt_all = wt[...]
    acc = dot(uft, wt_all[DU:2 * DU, :])
    acc += dot(ift, wt_all[2 * DU + DI:2 * DU + 2 * DI, :])
    acc += dot(cft, wt_all[2 * DU + 2 * DI:, :])
    au = wt_all[0:DU, :]
    z64 = jnp.zeros((DU, NUM_TASKS), f32)
    su0 = dot(uw[...], jnp.concatenate([au, z64], axis=0))
    su1 = dot(uw[...], jnp.concatenate([z64, au], axis=0))
    acc += jnp.where((uid[...] & 1) == 0, su0, su1)
    ai = wt_all[2 * DU:2 * DU + DI, :]
    z32 = jnp.zeros((DI, NUM_TASKS), f32)
    si = [dot(iw[...], jnp.concatenate(
        [z32] * q + [ai] + [z32] * (3 - q), axis=0)) for q in range(4)]
    iq = iid[...] & 3
    acc += jnp.where(iq < 2,
                     jnp.where(iq == 0, si[0], si[1]),
                     jnp.where(iq == 2, si[2], si[3]))
    out[...] = acc + bt[...]


def _dense_call(uf, itf, cf, uw, iw, uid, iid, wu, wi, wc, wt, bu, bi, bc,
                bt):
    grid = (B // BB,)
    row_blk = lambda w: pl.BlockSpec((BB, w), lambda i: (i, 0))
    full = lambda a: pl.BlockSpec(a.shape, lambda i: tuple(0 for _ in a.shape))
    return pl.pallas_call(
        _dense_body,
        grid=grid,
        in_specs=[
            row_blk(FU), row_blk(FI), row_blk(FC), row_blk(LANES),
            row_blk(LANES), row_blk(1), row_blk(1),
            full(wu), full(wi), full(wc), full(wt),
            full(bu), full(bi), full(bc), full(bt),
        ],
        out_specs=pl.BlockSpec((BB, NUM_TASKS), lambda i: (i, 0)),
        out_shape=jax.ShapeDtypeStruct((B, NUM_TASKS), jnp.float32),
    )(uf, itf, cf, uw, iw, uid, iid, wu, wi, wc, wt, bu, bi, bc, bt)


def kernel(user_id, user_features, item_id, item_features, cross_features,
           position, user_table, item_table, W_u, b_u, W_i, b_i, W_c, b_c,
           W_t, b_t):
    user_lines = user_table.reshape(-1, LANES)
    item_lines = item_table.reshape(-1, LANES)
    uw, iw = _sc_gather(user_lines, user_id, item_lines, item_id)
    return _dense_call(
        user_features, item_features, cross_features, uw, iw,
        user_id.reshape(B, 1), item_id.reshape(B, 1),
        W_u, W_i, W_c, W_t,
        b_u.reshape(1, DU), b_i.reshape(1, DI), b_c.reshape(1, CROSS_OUT),
        b_t.reshape(1, NUM_TASKS))


# speedup vs baseline: 4.4395x; 4.4395x over previous
import functools

import jax
import jax.numpy as jnp
from jax import lax
from jax.experimental import pallas as pl
from jax.experimental.pallas import tpu as pltpu
from jax.experimental.pallas import tpu_sc as plsc

B = 16384
DU = 64
DI = 32
FU = 128
FI = 128
FC = 128
NUM_TASKS = 4
CROSS_OUT = 128
VOCAB = 1000000

NC = 2
NS = 16
NW = NC * NS
BPW = B // NW
IDX_CHUNK = 128

VB = 8192
NBLK = -(-VOCAB // VB)
LB = VB // 128
NLINES = NBLK * LB

BB = 2048


def _pack_body(utabT, itabT, wt, *outs):
    au = wt[0:DU, :]
    ai = wt[2 * DU:2 * DU + DI, :]
    tn = (((0,), (0,)), ((), ()))
    pu = lax.dot_general(au, utabT[...], tn,
                         preferred_element_type=jnp.float32)
    pi = lax.dot_general(ai, itabT[...], tn,
                         preferred_element_type=jnp.float32)
    for t in range(NUM_TASKS):
        outs[t][...] = pu[t:t + 1, :].reshape(LB, 128)
        outs[NUM_TASKS + t][...] = pi[t:t + 1, :].reshape(LB, 128)


def _pack_call(utabT, itabT, wt):
    return pl.pallas_call(
        _pack_body,
        grid=(NBLK,),
        in_specs=[
            pl.BlockSpec((DU, VB), lambda i: (0, i)),
            pl.BlockSpec((DI, VB), lambda i: (0, i)),
            pl.BlockSpec(wt.shape, lambda i: (0, 0)),
        ],
        out_specs=[pl.BlockSpec((LB, 128), lambda i: (i, 0))] * 8,
        out_shape=[jax.ShapeDtypeStruct((NLINES, 128), jnp.float32)] * 8,
    )(utabT, itabT, wt)


def _sc_gather_body(pu0, pu1, pu2, pu3, pi0, pi1, pi2, pi3, uid, iid,
                    peT_out, ulv, ulane, ilv, ilane, buf, pe, sem):
    wid = lax.axis_index("s") * NC + lax.axis_index("c")
    base = wid * BPW
    pltpu.sync_copy(uid.at[pl.ds(base, BPW)], ulv)
    pltpu.sync_copy(iid.at[pl.ds(base, BPW)], ilv)
    for k in range(BPW // 16):
        s = pl.ds(k * 16, 16)
        uv = ulv[s]
        iv = ilv[s]
        ulane[s] = uv & 127
        ilane[s] = iv & 127
        ulv[s] = lax.shift_right_logical(uv, 7)
        ilv[s] = lax.shift_right_logical(iv, 7)

    iota16 = lax.iota(jnp.int32, 16)

    def extract(t, lane_ref, first):
        def chunk(k, carry):
            s = pl.ds(k * 16, 16)
            rows = jnp.full((16,), 16, jnp.int32) * k + iota16
            vals = plsc.load_gather(buf, [rows, lane_ref[s]])
            if first:
                pe[t, s] = vals
            else:
                pe[t, s] = pe[t, s] + vals
            return carry
        lax.fori_loop(0, BPW // 16, chunk, 0)

    def gather_lines(src, lref):
        for c in range(BPW // IDX_CHUNK):
            s = pl.ds(c * IDX_CHUNK, IDX_CHUNK)
            pltpu.async_copy(src.at[lref.at[s]], buf.at[s], sem)
        pltpu.make_async_copy(src.at[pl.ds(0, BPW)], buf, sem).wait()

    pus = (pu0, pu1, pu2, pu3)
    pis = (pi0, pi1, pi2, pi3)
    for t in range(NUM_TASKS):
        gather_lines(pus[t], ulv)
        extract(t, ulane, True)
        gather_lines(pis[t], ilv)
        extract(t, ilane, False)

    zeros16 = jnp.zeros((16,), jnp.float32)

    def zero_chunk(k, carry):
        s = pl.ds(k * 16, 16)
        for t in range(NUM_TASKS, 8):
            pe[t, s] = zeros16
        return carry
    lax.fori_loop(0, BPW // 16, zero_chunk, 0)

    pltpu.sync_copy(pe, peT_out.at[:, pl.ds(base, BPW)])


_sc_gather = pl.kernel(
    _sc_gather_body,
    out_type=jax.ShapeDtypeStruct((8, B), jnp.float32),
    mesh=plsc.VectorSubcoreMesh(core_axis_name="c", subcore_axis_name="s"),
    compiler_params=pltpu.CompilerParams(needs_layout_passes=False),
    scratch_types=[
        pltpu.VMEM((BPW,), jnp.int32),
        pltpu.VMEM((BPW,), jnp.int32),
        pltpu.VMEM((BPW,), jnp.int32),
        pltpu.VMEM((BPW,), jnp.int32),
        pltpu.VMEM((BPW, 128), jnp.float32),
        pltpu.VMEM((8, BPW), jnp.float32),
        pltpu.SemaphoreType.DMA,
    ],
)


def _dense_body(uf, itf, cf, peT, wu, wi, wc, wt, bu, bi, bc, bt, out):
    f32 = jnp.float32
    dot = functools.partial(jnp.dot, preferred_element_type=f32)
    uft = dot(uf[...], wu[...]) + bu[...]
    ift = dot(itf[...], wi[...]) + bi[...]
    cft = dot(cf[...], wc[...]) + bc[...]
    wt_all = wt[...]
    acc = dot(uft, wt_all[DU:2 * DU, :])
    acc += dot(ift, wt_all[2 * DU + DI:2 * DU + 2 * DI, :])
    acc += dot(cft, wt_all[2 * DU + 2 * DI:, :])
    tn = (((0,), (0,)), ((), ()))
    acc += lax.dot_general(peT[...], jnp.eye(8, NUM_TASKS, dtype=f32), tn,
                           preferred_element_type=f32)
    out[...] = acc + bt[...]


def _dense_call(uf, itf, cf, peT, wu, wi, wc, wt, bu, bi, bc, bt):
    grid = (B // BB,)
    row_blk = lambda w: pl.BlockSpec((BB, w), lambda i: (i, 0))
    full = lambda a: pl.BlockSpec(a.shape, lambda i: tuple(0 for _ in a.shape))
    return pl.pallas_call(
        _dense_body,
        grid=grid,
        in_specs=[
            row_blk(FU), row_blk(FI), row_blk(FC),
            pl.BlockSpec((8, BB), lambda i: (0, i)),
            full(wu), full(wi), full(wc), full(wt),
            full(bu), full(bi), full(bc), full(bt),
        ],
        out_specs=pl.BlockSpec((BB, NUM_TASKS), lambda i: (i, 0)),
        out_shape=jax.ShapeDtypeStruct((B, NUM_TASKS), jnp.float32),
    )(uf, itf, cf, peT, wu, wi, wc, wt, bu, bi, bc, bt)


def kernel(user_id, user_features, item_id, item_features, cross_features,
           position, user_table, item_table, W_u, b_u, W_i, b_i, W_c, b_c,
           W_t, b_t):
    lines = _pack_call(user_table.T, item_table.T, W_t)
    peT = _sc_gather(*lines, user_id, item_id)
    return _dense_call(
        user_features, item_features, cross_features, peT,
        W_u, W_i, W_c, W_t,
        b_u.reshape(1, DU), b_i.reshape(1, DI), b_c.reshape(1, CROSS_OUT),
        b_t.reshape(1, NUM_TASKS))


# speedup vs baseline: 5.8810x; 1.3247x over previous
import functools

import jax
import jax.numpy as jnp
from jax import lax
from jax.experimental import pallas as pl
from jax.experimental.pallas import tpu as pltpu
from jax.experimental.pallas import tpu_sc as plsc

B = 16384
DU = 64
DI = 32
FU = 128
FI = 128
FC = 128
NUM_TASKS = 4
CROSS_OUT = 128
VOCAB = 1000000

NC = 2
NS = 16
NW = NC * NS
BPW = B // NW
IDX_CHUNK = 128

VB = 16384
NBLK = -(-VOCAB // VB)
LB = VB // 128
NLINES = NBLK * LB
VFLAT = NLINES * 128

BB = 2048


def _pack_body(utabT, itabT, wt, *outs):
    au = wt[0:DU, :]
    ai = wt[2 * DU:2 * DU + DI, :]
    tn = (((0,), (0,)), ((), ()))
    pu = lax.dot_general(au, utabT[...], tn,
                         preferred_element_type=jnp.float32)
    pi = lax.dot_general(ai, itabT[...], tn,
                         preferred_element_type=jnp.float32)
    for t in range(NUM_TASKS):
        outs[t][...] = pu[t:t + 1, :].reshape(LB, 128)
        outs[NUM_TASKS + t][...] = pi[t:t + 1, :].reshape(LB, 128)


def _pack_call(utabT, itabT, wt):
    return pl.pallas_call(
        _pack_body,
        grid=(NBLK,),
        in_specs=[
            pl.BlockSpec((DU, VB), lambda i: (0, i)),
            pl.BlockSpec((DI, VB), lambda i: (0, i)),
            pl.BlockSpec(wt.shape, lambda i: (0, 0)),
        ],
        out_specs=[pl.BlockSpec((LB, 128), lambda i: (i, 0))] * 8,
        out_shape=[jax.ShapeDtypeStruct((NLINES, 128), jnp.float32)] * 8,
    )(utabT, itabT, wt)


def _sc_gather_body(pu0, pu1, pu2, pu3, pi0, pi1, pi2, pi3, uid, iid,
                    peT_out, ulv, ilv, gu, gi, pe, sem):
    wid = lax.axis_index("s") * NC + lax.axis_index("c")
    base = wid * BPW
    pltpu.sync_copy(uid.at[pl.ds(base, BPW)], ulv)
    pltpu.sync_copy(iid.at[pl.ds(base, BPW)], ilv)

    pus = (pu0, pu1, pu2, pu3)
    pis = (pi0, pi1, pi2, pi3)
    for t in range(NUM_TASKS):
        for c in range(BPW // IDX_CHUNK):
            s = pl.ds(c * IDX_CHUNK, IDX_CHUNK)
            pltpu.async_copy(pus[t].at[ulv.at[s]], gu.at[t, s], sem)
            pltpu.async_copy(pis[t].at[ilv.at[s]], gi.at[t, s], sem)
    for t in range(NUM_TASKS):
        pltpu.make_async_copy(pus[t].at[pl.ds(0, BPW)], gu.at[t], sem).wait()
        pltpu.make_async_copy(pis[t].at[pl.ds(0, BPW)], gi.at[t], sem).wait()

    def accum(k, carry):
        s = pl.ds(k * 16, 16)
        for t in range(NUM_TASKS):
            pe[t, s] = gu[t, s] + gi[t, s]
        return carry
    lax.fori_loop(0, BPW // 16, accum, 0)

    pltpu.sync_copy(pe, peT_out.at[:, pl.ds(base, BPW)])


_sc_gather = pl.kernel(
    _sc_gather_body,
    out_type=jax.ShapeDtypeStruct((NUM_TASKS, B), jnp.float32),
    mesh=plsc.VectorSubcoreMesh(core_axis_name="c", subcore_axis_name="s"),
    compiler_params=pltpu.CompilerParams(needs_layout_passes=False),
    scratch_types=[
        pltpu.VMEM((BPW,), jnp.int32),
        pltpu.VMEM((BPW,), jnp.int32),
        pltpu.VMEM((NUM_TASKS, BPW), jnp.float32),
        pltpu.VMEM((NUM_TASKS, BPW), jnp.float32),
        pltpu.VMEM((NUM_TASKS, BPW), jnp.float32),
        pltpu.SemaphoreType.DMA,
    ],
)


def _dense_body(uf, itf, cf, wu, wi, wc, wt, bu, bi, bc, out):
    f32 = jnp.float32
    dot = functools.partial(jnp.dot, preferred_element_type=f32)
    uft = dot(uf[...], wu[...]) + bu[...]
    ift = dot(itf[...], wi[...]) + bi[...]
    cft = dot(cf[...], wc[...]) + bc[...]
    wt_all = wt[...]
    acc = dot(uft, wt_all[DU:2 * DU, :])
    acc += dot(ift, wt_all[2 * DU + DI:2 * DU + 2 * DI, :])
    acc += dot(cft, wt_all[2 * DU + 2 * DI:, :])
    out[...] = acc


def _dense_call(uf, itf, cf, wu, wi, wc, wt, bu, bi, bc):
    grid = (B // BB,)
    row_blk = lambda w: pl.BlockSpec((BB, w), lambda i: (i, 0))
    full = lambda a: pl.BlockSpec(a.shape, lambda i: tuple(0 for _ in a.shape))
    return pl.pallas_call(
        _dense_body,
        grid=grid,
        in_specs=[
            row_blk(FU), row_blk(FI), row_blk(FC),
            full(wu), full(wi), full(wc), full(wt),
            full(bu), full(bi), full(bc),
        ],
        out_specs=pl.BlockSpec((BB, NUM_TASKS), lambda i: (i, 0)),
        out_shape=jax.ShapeDtypeStruct((B, NUM_TASKS), jnp.float32),
    )(uf, itf, cf, wu, wi, wc, wt, bu, bi, bc)


def _final_body(acc, peT, bt, out):
    tn = (((0,), (0,)), ((), ()))
    eye = jnp.eye(NUM_TASKS, NUM_TASKS, dtype=jnp.float32)
    out[...] = acc[...] + lax.dot_general(
        peT[...], eye, tn, preferred_element_type=jnp.float32) + bt[...]


def _final_call(acc, peT, bt):
    full = lambda a: pl.BlockSpec(a.shape, lambda i: tuple(0 for _ in a.shape))
    return pl.pallas_call(
        _final_body,
        grid=(1,),
        in_specs=[full(acc), full(peT), full(bt)],
        out_specs=pl.BlockSpec((B, NUM_TASKS), lambda i: (0, 0)),
        out_shape=jax.ShapeDtypeStruct((B, NUM_TASKS), jnp.float32),
    )(acc, peT, bt)


def kernel(user_id, user_features, item_id, item_features, cross_features,
           position, user_table, item_table, W_u, b_u, W_i, b_i, W_c, b_c,
           W_t, b_t):
    lines = _pack_call(user_table.T, item_table.T, W_t)
    flats = [a.reshape(VFLAT) for a in lines]
    peT = _sc_gather(*flats, user_id, item_id)
    acc = _dense_call(
        user_features, item_features, cross_features,
        W_u, W_i, W_c, W_t,
        b_u.reshape(1, DU), b_i.reshape(1, DI), b_c.reshape(1, CROSS_OUT))
    return _final_call(acc, peT, b_t.reshape(1, NUM_TASKS))


# speedup vs baseline: 5.9206x; 1.0067x over previous
import functools

import jax
import jax.numpy as jnp
from jax import lax
from jax.experimental import pallas as pl
from jax.experimental.pallas import tpu as pltpu
from jax.experimental.pallas import tpu_sc as plsc

B = 16384
DU = 64
DI = 32
FU = 128
FI = 128
FC = 128
NUM_TASKS = 4
CROSS_OUT = 128
VOCAB = 1000000

NC = 2
NS = 16
NW = NC * NS
BPW = B // NW
IDX_CHUNK = 128

VB = 32768
NBLK = -(-VOCAB // VB)
LB = VB // 128
NLINES = NBLK * LB
VFLAT = NLINES * 128

BB = 2048


def _pack_body(utabT, itabT, wt, *outs):
    au = wt[0:DU, :]
    ai = wt[2 * DU:2 * DU + DI, :]
    tn = (((0,), (0,)), ((), ()))
    pu = lax.dot_general(au, utabT[...], tn,
                         preferred_element_type=jnp.float32)
    pi = lax.dot_general(ai, itabT[...], tn,
                         preferred_element_type=jnp.float32)
    for t in range(NUM_TASKS):
        outs[t][...] = pu[t:t + 1, :].reshape(LB, 128)
        outs[NUM_TASKS + t][...] = pi[t:t + 1, :].reshape(LB, 128)


def _pack_call(utabT, itabT, wt):
    return pl.pallas_call(
        _pack_body,
        grid=(NBLK,),
        in_specs=[
            pl.BlockSpec((DU, VB), lambda i: (0, i)),
            pl.BlockSpec((DI, VB), lambda i: (0, i)),
            pl.BlockSpec(wt.shape, lambda i: (0, 0)),
        ],
        out_specs=[pl.BlockSpec((LB, 128), lambda i: (i, 0))] * 8,
        out_shape=[jax.ShapeDtypeStruct((NLINES, 128), jnp.float32)] * 8,
    )(utabT, itabT, wt)


def _sc_gather_body(pu0, pu1, pu2, pu3, pi0, pi1, pi2, pi3, uid, iid,
                    peT_out, ulv, ilv, gu, gi, pe, sem):
    wid = lax.axis_index("s") * NC + lax.axis_index("c")
    base = wid * BPW
    pltpu.sync_copy(uid.at[pl.ds(base, BPW)], ulv)
    pltpu.sync_copy(iid.at[pl.ds(base, BPW)], ilv)

    pus = (pu0, pu1, pu2, pu3)
    pis = (pi0, pi1, pi2, pi3)
    for t in range(NUM_TASKS):
        for c in range(BPW // IDX_CHUNK):
            s = pl.ds(c * IDX_CHUNK, IDX_CHUNK)
            pltpu.async_copy(pus[t].at[ulv.at[s]], gu.at[t, s], sem)
            pltpu.async_copy(pis[t].at[ilv.at[s]], gi.at[t, s], sem)
    for t in range(NUM_TASKS):
        pltpu.make_async_copy(pus[t].at[pl.ds(0, BPW)], gu.at[t], sem).wait()
        pltpu.make_async_copy(pis[t].at[pl.ds(0, BPW)], gi.at[t], sem).wait()

    def accum(k, carry):
        s = pl.ds(k * 16, 16)
        for t in range(NUM_TASKS):
            pe[t, s] = gu[t, s] + gi[t, s]
        return carry
    lax.fori_loop(0, BPW // 16, accum, 0)

    pltpu.sync_copy(pe, peT_out.at[:, pl.ds(base, BPW)])


_sc_gather = pl.kernel(
    _sc_gather_body,
    out_type=jax.ShapeDtypeStruct((NUM_TASKS, B), jnp.float32),
    mesh=plsc.VectorSubcoreMesh(core_axis_name="c", subcore_axis_name="s"),
    compiler_params=pltpu.CompilerParams(needs_layout_passes=False),
    scratch_types=[
        pltpu.VMEM((BPW,), jnp.int32),
        pltpu.VMEM((BPW,), jnp.int32),
        pltpu.VMEM((NUM_TASKS, BPW), jnp.float32),
        pltpu.VMEM((NUM_TASKS, BPW), jnp.float32),
        pltpu.VMEM((NUM_TASKS, BPW), jnp.float32),
        pltpu.SemaphoreType.DMA,
    ],
)


def _dense_body(uf, itf, cf, wu, wi, wc, wt, bu, bi, bc, out):
    f32 = jnp.float32
    dot = functools.partial(jnp.dot, preferred_element_type=f32)
    uft = dot(uf[...], wu[...]) + bu[...]
    ift = dot(itf[...], wi[...]) + bi[...]
    cft = dot(cf[...], wc[...]) + bc[...]
    wt_all = wt[...]
    acc = dot(uft, wt_all[DU:2 * DU, :])
    acc += dot(ift, wt_all[2 * DU + DI:2 * DU + 2 * DI, :])
    acc += dot(cft, wt_all[2 * DU + 2 * DI:, :])
    out[...] = acc


def _dense_call(uf, itf, cf, wu, wi, wc, wt, bu, bi, bc):
    grid = (B // BB,)
    row_blk = lambda w: pl.BlockSpec((BB, w), lambda i: (i, 0))
    full = lambda a: pl.BlockSpec(a.shape, lambda i: tuple(0 for _ in a.shape))
    return pl.pallas_call(
        _dense_body,
        grid=grid,
        in_specs=[
            row_blk(FU), row_blk(FI), row_blk(FC),
            full(wu), full(wi), full(wc), full(wt),
            full(bu), full(bi), full(bc),
        ],
        out_specs=pl.BlockSpec((BB, NUM_TASKS), lambda i: (i, 0)),
        out_shape=jax.ShapeDtypeStruct((B, NUM_TASKS), jnp.float32),
    )(uf, itf, cf, wu, wi, wc, wt, bu, bi, bc)


def _final_body(acc, peT, bt, out):
    tn = (((0,), (0,)), ((), ()))
    eye = jnp.eye(NUM_TASKS, NUM_TASKS, dtype=jnp.float32)
    out[...] = acc[...] + lax.dot_general(
        peT[...], eye, tn, preferred_element_type=jnp.float32) + bt[...]


def _final_call(acc, peT, bt):
    full = lambda a: pl.BlockSpec(a.shape, lambda i: tuple(0 for _ in a.shape))
    return pl.pallas_call(
        _final_body,
        grid=(1,),
        in_specs=[full(acc), full(peT), full(bt)],
        out_specs=pl.BlockSpec((B, NUM_TASKS), lambda i: (0, 0)),
        out_shape=jax.ShapeDtypeStruct((B, NUM_TASKS), jnp.float32),
    )(acc, peT, bt)


def kernel(user_id, user_features, item_id, item_features, cross_features,
           position, user_table, item_table, W_u, b_u, W_i, b_i, W_c, b_c,
           W_t, b_t):
    lines = _pack_call(user_table.T, item_table.T, W_t)
    flats = [a.reshape(VFLAT) for a in lines]
    peT = _sc_gather(*flats, user_id, item_id)
    acc = _dense_call(
        user_features, item_features, cross_features,
        W_u, W_i, W_c, W_t,
        b_u.reshape(1, DU), b_i.reshape(1, DI), b_c.reshape(1, CROSS_OUT))
    return _final_call(acc, peT, b_t.reshape(1, NUM_TASKS))


# speedup vs baseline: 6.0501x; 1.0219x over previous
import functools

import jax
import jax.numpy as jnp
from jax import lax
from jax.experimental import pallas as pl
from jax.experimental.pallas import tpu as pltpu
from jax.experimental.pallas import tpu_sc as plsc

B = 16384
DU = 64
DI = 32
FU = 128
FI = 128
FC = 128
NUM_TASKS = 4
CROSS_OUT = 128
VOCAB = 1000000

NC = 2
NS = 16
NW = NC * NS
BPW = B // NW
IDX_CHUNK = 128

VB = 32768
NBLK = -(-VOCAB // VB)
LB = VB // 128
NLINES = NBLK * LB
VFLAT = NLINES * 128

BB = 1024
NDB = B // BB


def _pack_body(utabT, itabT, wt, uf, itf, cf, wu, wi, wc, bu, bi, bc,
               *outs):
    au = wt[0:DU, :]
    ai = wt[2 * DU:2 * DU + DI, :]
    tn = (((0,), (0,)), ((), ()))
    pu = lax.dot_general(au, utabT[...], tn,
                         preferred_element_type=jnp.float32)
    pi = lax.dot_general(ai, itabT[...], tn,
                         preferred_element_type=jnp.float32)
    for t in range(NUM_TASKS):
        outs[t][...] = pu[t:t + 1, :].reshape(LB, 128)
        outs[NUM_TASKS + t][...] = pi[t:t + 1, :].reshape(LB, 128)

    i = pl.program_id(0)

    @pl.when(i < NDB)
    def _dense():
        f32 = jnp.float32
        dot = functools.partial(jnp.dot, preferred_element_type=f32)
        uft = dot(uf[...], wu[...]) + bu[...]
        ift = dot(itf[...], wi[...]) + bi[...]
        cft = dot(cf[...], wc[...]) + bc[...]
        wt_all = wt[...]
        acc = dot(uft, wt_all[DU:2 * DU, :])
        acc += dot(ift, wt_all[2 * DU + DI:2 * DU + 2 * DI, :])
        acc += dot(cft, wt_all[2 * DU + 2 * DI:, :])
        outs[8][...] = lax.transpose(acc, (1, 0))


def _pack_call(utabT, itabT, wt, uf, itf, cf, wu, wi, wc, bu, bi, bc):
    dense_i = lambda i: (jnp.minimum(i, NDB - 1), 0)
    full = lambda a: pl.BlockSpec(a.shape, lambda i: tuple(0 for _ in a.shape))
    return pl.pallas_call(
        _pack_body,
        grid=(NBLK,),
        in_specs=[
            pl.BlockSpec((DU, VB), lambda i: (0, i)),
            pl.BlockSpec((DI, VB), lambda i: (0, i)),
            full(wt),
            pl.BlockSpec((BB, FU), dense_i),
            pl.BlockSpec((BB, FI), dense_i),
            pl.BlockSpec((BB, FC), dense_i),
            full(wu), full(wi), full(wc),
            full(bu), full(bi), full(bc),
        ],
        out_specs=[pl.BlockSpec((LB, 128), lambda i: (i, 0))] * 8 + [
            pl.BlockSpec((NUM_TASKS, BB),
                         lambda i: (0, jnp.minimum(i, NDB - 1)))],
        out_shape=[jax.ShapeDtypeStruct((NLINES, 128), jnp.float32)] * 8 + [
            jax.ShapeDtypeStruct((NUM_TASKS, B), jnp.float32)],
    )(utabT, itabT, wt, uf, itf, cf, wu, wi, wc, bu, bi, bc)


def _sc_gather_body(pu0, pu1, pu2, pu3, pi0, pi1, pi2, pi3, uid, iid,
                    peT_out, ulv, ilv, gu, gi, pe, sem):
    wid = lax.axis_index("s") * NC + lax.axis_index("c")
    base = wid * BPW
    pltpu.sync_copy(uid.at[pl.ds(base, BPW)], ulv)
    pltpu.sync_copy(iid.at[pl.ds(base, BPW)], ilv)

    pus = (pu0, pu1, pu2, pu3)
    pis = (pi0, pi1, pi2, pi3)
    for t in range(NUM_TASKS):
        for c in range(BPW // IDX_CHUNK):
            s = pl.ds(c * IDX_CHUNK, IDX_CHUNK)
            pltpu.async_copy(pus[t].at[ulv.at[s]], gu.at[t, s], sem)
            pltpu.async_copy(pis[t].at[ilv.at[s]], gi.at[t, s], sem)
    for t in range(NUM_TASKS):
        pltpu.make_async_copy(pus[t].at[pl.ds(0, BPW)], gu.at[t], sem).wait()
        pltpu.make_async_copy(pis[t].at[pl.ds(0, BPW)], gi.at[t], sem).wait()

    def accum(k, carry):
        s = pl.ds(k * 16, 16)
        for t in range(NUM_TASKS):
            pe[t, s] = gu[t, s] + gi[t, s]
        return carry
    lax.fori_loop(0, BPW // 16, accum, 0)

    pltpu.sync_copy(pe, peT_out.at[:, pl.ds(base, BPW)])


_sc_gather = pl.kernel(
    _sc_gather_body,
    out_type=jax.ShapeDtypeStruct((NUM_TASKS, B), jnp.float32),
    mesh=plsc.VectorSubcoreMesh(core_axis_name="c", subcore_axis_name="s"),
    compiler_params=pltpu.CompilerParams(needs_layout_passes=False),
    scratch_types=[
        pltpu.VMEM((BPW,), jnp.int32),
        pltpu.VMEM((BPW,), jnp.int32),
        pltpu.VMEM((NUM_TASKS, BPW), jnp.float32),
        pltpu.VMEM((NUM_TASKS, BPW), jnp.float32),
        pltpu.VMEM((NUM_TASKS, BPW), jnp.float32),
        pltpu.SemaphoreType.DMA,
    ],
)


FB = 2048


def _final_body(accT, peT, bt, out):
    tn = (((0,), (0,)), ((), ()))
    eye = jnp.eye(NUM_TASKS, NUM_TASKS, dtype=jnp.float32)
    out[...] = lax.dot_general(
        accT[...] + peT[...], eye, tn,
        preferred_element_type=jnp.float32) + bt[...]


def _final_call(accT, peT, bt):
    full = lambda a: pl.BlockSpec(a.shape, lambda i: tuple(0 for _ in a.shape))
    return pl.pallas_call(
        _final_body,
        grid=(B // FB,),
        in_specs=[
            pl.BlockSpec((NUM_TASKS, FB), lambda i: (0, i)),
            pl.BlockSpec((NUM_TASKS, FB), lambda i: (0, i)),
            full(bt),
        ],
        out_specs=pl.BlockSpec((FB, NUM_TASKS), lambda i: (i, 0)),
        out_shape=jax.ShapeDtypeStruct((B, NUM_TASKS), jnp.float32),
    )(accT, peT, bt)


def kernel(user_id, user_features, item_id, item_features, cross_features,
           position, user_table, item_table, W_u, b_u, W_i, b_i, W_c, b_c,
           W_t, b_t):
    packed = _pack_call(
        user_table.T, item_table.T, W_t,
        user_features, item_features, cross_features,
        W_u, W_i, W_c,
        b_u.reshape(1, DU), b_i.reshape(1, DI), b_c.reshape(1, CROSS_OUT))
    lines, accT = packed[:8], packed[8]
    flats = [a.reshape(VFLAT) for a in lines]
    peT = _sc_gather(*flats, user_id, item_id)
    return _final_call(accT, peT, b_t.reshape(1, NUM_TASKS))
